# baseline (speedup 1.0000x reference)
import jax
import jax.numpy as jnp
from jax.experimental import pallas as pl
from jax.experimental.pallas import tpu as pltpu

_CHUNK = 4096
_LIN_BLOCK = 2000


def _edge_kernel(src_ref, dst_ref, rel_ref, h_ref, rin_ref, out_ref):
    @pl.when(pl.program_id(0) == 0)
    def _init():
        out_ref[...] = jnp.zeros_like(out_ref)

    def body(e, carry):
        s = src_ref[e]
        d = dst_ref[e]
        r = rel_ref[e]
        msg = h_ref[pl.ds(s, 1), :] * rin_ref[pl.ds(r, 1), :]
        out_ref[pl.ds(d, 1), :] = out_ref[pl.ds(d, 1), :] + msg
        return carry

    jax.lax.fori_loop(0, src_ref.shape[0], body, 0)


def _rspmm_b(h2, rin2, src, dst, rel):
    nchunk = src.shape[0] // _CHUNK
    return pl.pallas_call(
        _edge_kernel,
        grid=(nchunk,),
        in_specs=[
            pl.BlockSpec((_CHUNK,), lambda i: (i,), memory_space=pltpu.SMEM),
            pl.BlockSpec((_CHUNK,), lambda i: (i,), memory_space=pltpu.SMEM),
            pl.BlockSpec((_CHUNK,), lambda i: (i,), memory_space=pltpu.SMEM),
            pl.BlockSpec(h2.shape, lambda i: (0, 0)),
            pl.BlockSpec(rin2.shape, lambda i: (0, 0)),
        ],
        out_specs=pl.BlockSpec(h2.shape, lambda i: (0, 0)),
        out_shape=jax.ShapeDtypeStruct(h2.shape, jnp.float32),
    )(src, dst, rel, h2, rin2)


def _lin_kernel(x_ref, w_ref, b_ref, o_ref):
    y = jnp.dot(x_ref[...], w_ref[...], preferred_element_type=jnp.float32)
    o_ref[...] = jnp.maximum(y + b_ref[...], 0.0)


def _lin_relu(x, w, bias):
    m, d = x.shape
    return pl.pallas_call(
        _lin_kernel,
        grid=(m // _LIN_BLOCK,),
        in_specs=[
            pl.BlockSpec((_LIN_BLOCK, d), lambda i: (i, 0)),
            pl.BlockSpec((d, d), lambda i: (0, 0)),
            pl.BlockSpec((1, d), lambda i: (0, 0)),
        ],
        out_specs=pl.BlockSpec((_LIN_BLOCK, d), lambda i: (i, 0)),
        out_shape=jax.ShapeDtypeStruct((m, d), jnp.float32),
    )(x, w, bias.reshape(1, d))


def kernel(head, tail, edge_src, edge_dst, edge_rel, ent_kg, rel_kg0, rel_kg1,
           lin_w0, lin_b0, lin_w1, lin_b1, rel_lin_w0, rel_lin_b0, rel_lin_w1,
           rel_lin_b1, attn_w0, attn_b0, attn_w1, attn_b1):
    n_ent, d = ent_kg.shape
    bsz = head.shape[0]
    nrel = rel_kg0.shape[0]

    head_embed = ent_kg[head]
    tail_embed = ent_kg[tail]
    ht = jnp.concatenate([head_embed, tail_embed], axis=-1)

    rel_kgs = (rel_kg0, rel_kg1)
    lin_ws = (lin_w0, lin_w1)
    lin_bs = (lin_b0, lin_b1)
    rel_lins = ((rel_lin_w0, rel_lin_b0), (rel_lin_w1, rel_lin_b1))
    attns = ((attn_w0, attn_b0), (attn_w1, attn_b1))

    e = edge_src.shape[0]
    pad = (-e) % _CHUNK
    src = jnp.concatenate(
        [edge_src.astype(jnp.int32), jnp.zeros((pad,), jnp.int32)])
    dst = jnp.concatenate(
        [edge_dst.astype(jnp.int32), jnp.zeros((pad,), jnp.int32)])
    rel = jnp.concatenate(
        [edge_rel.astype(jnp.int32), jnp.full((pad,), nrel, jnp.int32)])

    def propagate(init_idx, init_embed):
        hid = jnp.zeros((n_ent, bsz, d), jnp.float32)
        hid = hid.at[init_idx, jnp.arange(bsz)].set(init_embed)
        for l in range(2):
            rw = jax.nn.sigmoid(
                jax.nn.relu(ht @ rel_lins[l][0] + rel_lins[l][1])
                @ attns[l][0] + attns[l][1])
            rin = jnp.transpose(rw[:, :, None] * rel_kgs[l][None], (1, 0, 2))
            rin = jnp.concatenate(
                [rin, jnp.zeros((1, bsz, d), jnp.float32)], axis=0)
            aggs = [_rspmm_b(hid[:, b, :], rin[:, b, :], src, dst, rel)
                    for b in range(bsz)]
            agg = jnp.stack(aggs, axis=1)
            hid = _lin_relu(agg.reshape(n_ent * bsz, d),
                            lin_ws[l], lin_bs[l]).reshape(n_ent, bsz, d)
        return hid

    tail_hid = propagate(head, head_embed)[tail, jnp.arange(bsz)]
    head_hid = propagate(tail, tail_embed)[head, jnp.arange(bsz)]
    return jnp.concatenate([head_embed, tail_embed, head_hid, tail_hid],
                           axis=1)
